# baseline (speedup 1.0000x reference)
import functools
import jax
import jax.numpy as jnp
import numpy as np
from jax import lax
from jax.experimental import pallas as pl
from jax.experimental.pallas import tpu as pltpu
from jax.experimental.pallas import tpu_sc as plsc

_V = 60000
_EPS = 1e-5


def _sc_gather(table, idx, ch):
    info = plsc.get_sparse_core_info()
    nc, ns = info.num_cores, info.num_subcores
    nw = nc * ns
    b = idx.shape[0]
    d = table.shape[1]
    b_per_w = b // nw
    n_chunks = b_per_w // ch
    mesh = plsc.VectorSubcoreMesh(core_axis_name="c", subcore_axis_name="s")

    @functools.partial(
        pl.kernel, mesh=mesh,
        out_type=jax.ShapeDtypeStruct((b, d), jnp.float32),
        scratch_types=[
            pltpu.VMEM((ch,), jnp.int32),
            pltpu.VMEM((ch, d), jnp.float32),
            pltpu.SemaphoreType.DMA,
        ],
    )
    def k(table_hbm, idx_hbm, out_hbm, idx_v, rows_v, sem):
        wid = lax.axis_index("s") * nc + lax.axis_index("c")
        base = wid * b_per_w
        for i in range(n_chunks):
            off = base + i * ch
            pltpu.sync_copy(idx_hbm.at[pl.ds(off, ch)], idx_v)
            pltpu.async_copy(table_hbm.at[idx_v], rows_v, sem).wait()
            pltpu.sync_copy(rows_v, out_hbm.at[pl.ds(off, ch)])

    return k(table, idx)


def _tril(blk):
    ri = lax.broadcasted_iota(jnp.int32, (blk, blk), 0)
    ci = lax.broadcasted_iota(jnp.int32, (blk, blk), 1)
    return jnp.where(ri >= ci, 1.0, 0.0).astype(jnp.float32)


def _cumsum_body(x_ref, o_ref, carry_ref):
    @pl.when(pl.program_id(0) == 0)
    def _():
        carry_ref[...] = jnp.zeros_like(carry_ref)

    blk = x_ref[...]
    tril = _tril(blk.shape[0])
    c = jnp.dot(tril, blk, preferred_element_type=jnp.float32,
                precision=lax.Precision.HIGHEST) + carry_ref[0:1, :]
    o_ref[...] = c
    carry_ref[0:1, :] = c[-1:, :]


def _cumsum(x, blk):
    n, d = x.shape
    return pl.pallas_call(
        _cumsum_body,
        grid=(n // blk,),
        in_specs=[pl.BlockSpec((blk, d), lambda i: (i, 0))],
        out_specs=pl.BlockSpec((blk, d), lambda i: (i, 0)),
        out_shape=jax.ShapeDtypeStruct((n, d), jnp.float32),
        scratch_shapes=[pltpu.VMEM((8, d), jnp.float32)],
    )(x)


def _vox_stats_body(a_ref, b_ref, o_ref):
    s = b_ref[...] - a_ref[...]
    cnt = s[:, 3:4]
    mean = s[:, 0:3] / jnp.maximum(cnt, 1.0)
    o_ref[...] = jnp.concatenate(
        [mean, cnt, jnp.zeros((s.shape[0], 124), jnp.float32)], axis=1)


def _vox_feat_body(a_ref, b_ref, st_ref, t_ref, o_ref):
    s = b_ref[:, 0:64] - a_ref[:, 0:64]
    cnt = st_ref[:, 3:4]
    nonempty = jnp.where(cnt > 0.0, 1.0, 0.0)
    o_ref[...] = (s / jnp.maximum(cnt, 1.0) + t_ref[0:1, :]) * nonempty


def _feat_body(p_ref, m_ref, f_ref, s_ref, *, blk, n):
    step = pl.program_id(0)

    @pl.when(step == 0)
    def _():
        s_ref[...] = jnp.zeros_like(s_ref)

    p = p_ref[...]
    xyz = p[:, 0:3]
    pts4 = p[:, 0:4]
    nrm = p[:, 4:7]
    grid = p[:, 8:11]
    mean = m_ref[:, 0:3]
    neg_low = jnp.concatenate([jnp.full((blk, 1), 50.0, jnp.float32),
                               jnp.full((blk, 1), 50.0, jnp.float32),
                               jnp.full((blk, 1), 4.0, jnp.float32)], axis=1)
    ctp_s = xyz - grid * 0.1
    mask = (lax.broadcasted_iota(jnp.int32, (blk, 1), 0) + step * blk) < n
    fs = jnp.concatenate(
        [pts4, xyz - mean, ctp_s, nrm,
         jnp.zeros((blk, 3), jnp.float32)], axis=1)
    fs = jnp.where(mask, fs, 0.0)
    shift = jnp.concatenate([jnp.zeros((blk, 7), jnp.float32), neg_low,
                             jnp.zeros((blk, 6), jnp.float32)], axis=1)
    f_ref[...] = jnp.where(mask, fs + shift, 0.0)
    ftf = lax.dot_general(fs, fs, (((0,), (0,)), ((), ())),
                          preferred_element_type=jnp.float32,
                          precision=lax.Precision.HIGHEST)
    colsum = jnp.sum(fs, axis=0, keepdims=True)
    s_ref[0:16, :] += ftf
    s_ref[16:17, :] += colsum


def _mlp12_body(f_ref, w1_ref, b1_ref, w2_ref, b2_ref, x2_ref, s_ref, *, blk, n):
    step = pl.program_id(0)

    @pl.when(step == 0)
    def _():
        s_ref[...] = jnp.zeros_like(s_ref)

    f = f_ref[...]
    x1 = jnp.dot(f, w1_ref[...], preferred_element_type=jnp.float32,
                 precision=lax.Precision.HIGHEST) + b1_ref[0:1, :]
    h1 = jnp.maximum(x1, 0.0)
    x2 = jnp.dot(h1, w2_ref[...], preferred_element_type=jnp.float32,
                 precision=lax.Precision.HIGHEST) + b2_ref[0:1, :]
    mask = (lax.broadcasted_iota(jnp.int32, (blk, 1), 0) + step * blk) < n
    x2 = jnp.where(mask, x2, 0.0)
    x2_ref[...] = x2
    s_ref[0:1, :] += jnp.sum(x2, axis=0, keepdims=True)
    s_ref[1:2, :] += jnp.sum(x2 * x2, axis=0, keepdims=True)


def _h2sum_body(x2_ref, a2_ref, c2_ref, s_ref, *, blk, n):
    step = pl.program_id(0)

    @pl.when(step == 0)
    def _():
        s_ref[...] = jnp.zeros_like(s_ref)

    h2 = jnp.maximum(x2_ref[...] * a2_ref[0:1, :] + c2_ref[0:1, :], 0.0)
    mask = (lax.broadcasted_iota(jnp.int32, (blk, 1), 0) + step * blk) < n
    h2 = jnp.where(mask, h2, 0.0)
    s_ref[0:1, :] += jnp.sum(h2, axis=0, keepdims=True)


def _mlp3cs_body(x2_ref, a2_ref, c2_ref, w3_ref, b3_ref, t_ref, o_ref, carry_ref,
                 *, blk, n):
    step = pl.program_id(0)

    @pl.when(step == 0)
    def _():
        carry_ref[...] = jnp.zeros_like(carry_ref)

    x2 = x2_ref[...]
    h2 = jnp.maximum(x2 * a2_ref[0:1, :] + c2_ref[0:1, :], 0.0)
    h3 = (jnp.dot(h2, w3_ref[...], preferred_element_type=jnp.float32,
                  precision=lax.Precision.HIGHEST)
          + b3_ref[0:1, :] - t_ref[0:1, :])
    mask = (lax.broadcasted_iota(jnp.int32, (blk, 1), 0) + step * blk) < n
    h3 = jnp.where(mask, h3, 0.0)
    c = jnp.dot(_tril(blk), h3, preferred_element_type=jnp.float32,
                precision=lax.Precision.HIGHEST) + carry_ref[0:1, :]
    o_ref[...] = c
    carry_ref[0:1, :] = c[-1:, :]


def _full_spec(r, c):
    return pl.BlockSpec((r, c), lambda i: (0, 0))


def kernel(points, normal, grid_ind, coors_inv, gamma0, beta0, W1, b1,
           gamma1, beta1, W2, b2, gamma2, beta2, W3, b3):
    n = points.shape[0]
    blk = 512
    ch = 512
    n_pad = ((n + 32 * ch - 1) // (32 * ch)) * (32 * ch)
    v_pad = ((_V + 32 * ch - 1) // (32 * ch)) * (32 * ch)

    ids = coors_inv.astype(jnp.int32)
    a_idx = jnp.searchsorted(ids, jnp.arange(_V, dtype=jnp.int32),
                             side="left").astype(jnp.int32)
    b_idx = jnp.searchsorted(ids, jnp.arange(_V, dtype=jnp.int32),
                             side="right").astype(jnp.int32)
    a_pad = jnp.pad(a_idx, (0, v_pad - _V))
    b_pad = jnp.pad(b_idx, (0, v_pad - _V))
    ids_pad = jnp.pad(ids, (0, n_pad - n))

    p16 = jnp.concatenate([
        points,
        normal, jnp.zeros((n, 1), jnp.float32),
        grid_ind.astype(jnp.float32), jnp.zeros((n, 5), jnp.float32),
    ], axis=1)
    p16 = jnp.pad(p16, ((0, n_pad - n), (0, 0)))

    x4 = jnp.concatenate([
        points[:, 0:3], jnp.ones((n, 1), jnp.float32),
        jnp.zeros((n, 124), jnp.float32)], axis=1)
    x4 = jnp.pad(x4, ((0, n_pad - n), (0, 0)))
    c4 = _cumsum(x4, blk)
    c4z = jnp.concatenate([jnp.zeros((1, 128), jnp.float32), c4], axis=0)

    ga4 = _sc_gather(c4z, a_pad, ch)
    gb4 = _sc_gather(c4z, b_pad, ch)
    vox16 = pl.pallas_call(
        _vox_stats_body,
        grid=(v_pad // blk,),
        in_specs=[pl.BlockSpec((blk, 128), lambda i: (i, 0))] * 2,
        out_specs=pl.BlockSpec((blk, 128), lambda i: (i, 0)),
        out_shape=jax.ShapeDtypeStruct((v_pad, 128), jnp.float32),
    )(ga4, gb4)

    m16 = _sc_gather(vox16, ids_pad, ch)

    f_out, s_out = pl.pallas_call(
        functools.partial(_feat_body, blk=blk, n=n),
        grid=(n_pad // blk,),
        in_specs=[pl.BlockSpec((blk, 16), lambda i: (i, 0)),
                  pl.BlockSpec((blk, 128), lambda i: (i, 0))],
        out_specs=[pl.BlockSpec((blk, 16), lambda i: (i, 0)),
                   pl.BlockSpec((24, 16), lambda i: (0, 0))],
        out_shape=[jax.ShapeDtypeStruct((n_pad, 16), jnp.float32),
                   jax.ShapeDtypeStruct((24, 16), jnp.float32)],
    )(p16, m16)

    nf = jnp.float32(n)
    shift16 = jnp.zeros((16,), jnp.float32).at[7].set(50.0).at[8].set(50.0).at[9].set(4.0)
    sum_f = s_out[16, :]
    ftf = s_out[0:16, :]
    m0s = sum_f / nf
    cov = ftf / nf - m0s[:, None] * m0s[None, :]
    m0 = m0s + shift16
    v0 = jnp.diag(cov)
    inv_s0 = 1.0 / jnp.sqrt(v0 + _EPS)
    g16 = jnp.pad(gamma0, (0, 3))
    bt16 = jnp.pad(beta0, (0, 3))
    scale0 = g16 * inv_s0
    w1_16 = jnp.pad(W1, ((0, 3), (0, 0)))
    w1e = scale0[:, None] * w1_16
    b1e = b1 + jnp.dot(bt16 - m0 * scale0, w1_16, precision=lax.Precision.HIGHEST)
    m1 = jnp.dot(m0, w1e, precision=lax.Precision.HIGHEST) + b1e
    v1 = jnp.einsum("kj,kl,lj->j", w1e, cov, w1e, precision=lax.Precision.HIGHEST)
    a1 = gamma1 / jnp.sqrt(v1 + _EPS)
    c1 = beta1 - m1 * a1
    w1p = w1e * a1[None, :]
    b1p = (b1e * a1 + c1)[None, :]

    x2_out, s2_out = pl.pallas_call(
        functools.partial(_mlp12_body, blk=blk, n=n),
        grid=(n_pad // blk,),
        in_specs=[pl.BlockSpec((blk, 16), lambda i: (i, 0)),
                  _full_spec(16, 64), _full_spec(1, 64),
                  _full_spec(64, 64), _full_spec(1, 64)],
        out_specs=[pl.BlockSpec((blk, 64), lambda i: (i, 0)),
                   pl.BlockSpec((8, 64), lambda i: (0, 0))],
        out_shape=[jax.ShapeDtypeStruct((n_pad, 64), jnp.float32),
                   jax.ShapeDtypeStruct((8, 64), jnp.float32)],
    )(f_out, w1p, b1p, W2, b2[None, :])

    m2 = s2_out[0, :] / nf
    v2 = s2_out[1, :] / nf - m2 * m2
    a2 = gamma2 / jnp.sqrt(v2 + _EPS)
    c2 = beta2 - m2 * a2

    h2s = pl.pallas_call(
        functools.partial(_h2sum_body, blk=blk, n=n),
        grid=(n_pad // blk,),
        in_specs=[pl.BlockSpec((blk, 64), lambda i: (i, 0)),
                  _full_spec(1, 64), _full_spec(1, 64)],
        out_specs=pl.BlockSpec((8, 64), lambda i: (0, 0)),
        out_shape=jax.ShapeDtypeStruct((8, 64), jnp.float32),
    )(x2_out, a2[None, :], c2[None, :])
    t = jnp.dot(h2s[0, :], W3, precision=lax.Precision.HIGHEST) / nf + b3

    c64 = pl.pallas_call(
        functools.partial(_mlp3cs_body, blk=blk, n=n),
        grid=(n_pad // blk,),
        in_specs=[pl.BlockSpec((blk, 64), lambda i: (i, 0)),
                  _full_spec(1, 64), _full_spec(1, 64),
                  _full_spec(64, 128), _full_spec(1, 128), _full_spec(1, 128)],
        out_specs=pl.BlockSpec((blk, 128), lambda i: (i, 0)),
        out_shape=jax.ShapeDtypeStruct((n_pad, 128), jnp.float32),
        scratch_shapes=[pltpu.VMEM((8, 128), jnp.float32)],
    )(x2_out, a2[None, :], c2[None, :],
      jnp.pad(W3, ((0, 0), (0, 64))), jnp.pad(b3, (0, 64))[None, :],
      jnp.pad(t, (0, 64))[None, :])

    c64z = jnp.concatenate([jnp.zeros((1, 128), jnp.float32), c64], axis=0)
    ga64 = _sc_gather(c64z, a_pad, ch)
    gb64 = _sc_gather(c64z, b_pad, ch)

    feats = pl.pallas_call(
        _vox_feat_body,
        grid=(v_pad // blk,),
        in_specs=[pl.BlockSpec((blk, 128), lambda i: (i, 0)),
                  pl.BlockSpec((blk, 128), lambda i: (i, 0)),
                  pl.BlockSpec((blk, 128), lambda i: (i, 0)),
                  _full_spec(1, 64)],
        out_specs=pl.BlockSpec((blk, 64), lambda i: (i, 0)),
        out_shape=jax.ShapeDtypeStruct((v_pad, 64), jnp.float32),
    )(ga64, gb64, vox16, t[None, :])

    return feats[:_V]
